# Initial kernel scaffold; baseline (speedup 1.0000x reference)
import jax
import jax.numpy as jnp
from jax.experimental import pallas as pl


def kernel(dst_xyz, dst_feat, src_xyz, src_feat, W1, b1, g1, be1, W2, b2, g2, be2):
    raise NotImplementedError("write your pallas kernel here")


# speedup vs baseline: 19.6140x; 19.6140x over previous
import jax
import jax.numpy as jnp
from jax.experimental import pallas as pl
from jax.experimental.pallas import tpu as pltpu

_B, _N, _M = 4, 8192, 2048
_C11, _C12 = 32, 64
_CH = 64
_BN1 = 256
_BN2 = 2048
_NPTS = float(_B * _N)
_EPS_BN = 1e-5
_EPS_D = 1e-8


def _knn_interp_l1_kernel(dst_xyz_ref, x2_ref, dst_feat_ref, src_xyzT_ref,
                          s2_ref, src_feat_ref,
                          W1aT_ref, W1bT_ref, b1_ref, y1_ref, stats_ref):
    m = jnp.dot(dst_xyz_ref[0], src_xyzT_ref[0],
                preferred_element_type=jnp.float32)
    d = -2.0 * m
    d = d + x2_ref[0]
    d = d + s2_ref[0]

    iota = jax.lax.broadcasted_iota(jnp.int32, d.shape, 1)
    w_acc = jnp.zeros_like(d)
    rsum = jnp.zeros((d.shape[0], 1), jnp.float32)
    dd = d
    for _ in range(3):
        mval = jnp.min(dd, axis=1, keepdims=True)
        midx = jnp.min(jnp.where(dd == mval, iota, _M),
                       axis=1, keepdims=True)
        hit = iota == midx
        r = 1.0 / (mval + _EPS_D)
        w_acc = w_acc + jnp.where(hit, r, 0.0)
        rsum = rsum + r
        dd = jnp.where(hit, jnp.float32(jnp.inf), dd)
    wn = w_acc / rsum

    interp = jnp.dot(wn, src_feat_ref[0],
                     preferred_element_type=jnp.float32,
                     precision=jax.lax.Precision.HIGHEST)
    y1 = (jnp.dot(dst_feat_ref[0], W1aT_ref[...],
                  preferred_element_type=jnp.float32)
          + jnp.dot(interp, W1bT_ref[...],
                    preferred_element_type=jnp.float32)
          + b1_ref[...])
    y1_ref[...] = y1[None]

    first = jnp.logical_and(pl.program_id(0) == 0, pl.program_id(1) == 0)

    @pl.when(first)
    def _init():
        stats_ref[...] = jnp.zeros_like(stats_ref)

    su = jnp.sum(y1, axis=0, keepdims=True)
    sq = jnp.sum(y1 * y1, axis=0, keepdims=True)
    stats_ref[...] += jnp.concatenate(
        [su, sq, jnp.zeros((6, _CH), jnp.float32)], axis=0)


def _bn_relu_l2_kernel(y1_ref, stats1_ref, g1_ref, be1_ref, W2T_ref, b2_ref,
                       y2_ref, stats2_ref):
    st = stats1_ref[...]
    mean = st[0:1, :] / _NPTS
    var = st[1:2, :] / _NPTS - mean * mean
    y = y1_ref[...]
    h = jnp.maximum(g1_ref[...] * (y - mean) * jax.lax.rsqrt(var + _EPS_BN)
                    + be1_ref[...], 0.0)
    y2 = jnp.dot(h, W2T_ref[...],
                 preferred_element_type=jnp.float32) + b2_ref[...]
    y2_ref[...] = y2

    @pl.when(pl.program_id(0) == 0)
    def _init():
        stats2_ref[...] = jnp.zeros_like(stats2_ref)

    su = jnp.sum(y2, axis=0, keepdims=True)
    sq = jnp.sum(y2 * y2, axis=0, keepdims=True)
    stats2_ref[...] += jnp.concatenate(
        [su, sq, jnp.zeros((6, _CH), jnp.float32)], axis=0)


def _bn_relu_kernel(y2_ref, stats2_ref, g2_ref, be2_ref, out_ref):
    st = stats2_ref[...]
    mean = st[0:1, :] / _NPTS
    var = st[1:2, :] / _NPTS - mean * mean
    y = y2_ref[...]
    out_ref[...] = jnp.maximum(
        g2_ref[...] * (y - mean) * jax.lax.rsqrt(var + _EPS_BN) + be2_ref[...],
        0.0)


def kernel(dst_xyz, dst_feat, src_xyz, src_feat, W1, b1, g1, be1, W2, b2, g2, be2):
    src_xyzT = jnp.transpose(src_xyz, (0, 2, 1))
    x2 = jnp.sum(dst_xyz ** 2, axis=-1, keepdims=True)
    s2 = jnp.sum(src_xyz ** 2, axis=-1)[:, None, :]
    W1aT = jnp.transpose(W1[:, :_C11])
    W1bT = jnp.transpose(W1[:, _C11:])
    W2T = jnp.transpose(W2)
    b1r = b1.reshape(1, _CH)
    b2r = b2.reshape(1, _CH)
    g1r = g1.reshape(1, _CH)
    be1r = be1.reshape(1, _CH)
    g2r = g2.reshape(1, _CH)
    be2r = be2.reshape(1, _CH)

    nb1 = _N // _BN1
    y1, stats1 = pl.pallas_call(
        _knn_interp_l1_kernel,
        grid=(_B, nb1),
        in_specs=[
            pl.BlockSpec((1, _BN1, 3), lambda b, i: (b, i, 0)),
            pl.BlockSpec((1, _BN1, 1), lambda b, i: (b, i, 0)),
            pl.BlockSpec((1, _BN1, _C11), lambda b, i: (b, i, 0)),
            pl.BlockSpec((1, 3, _M), lambda b, i: (b, 0, 0)),
            pl.BlockSpec((1, 1, _M), lambda b, i: (b, 0, 0)),
            pl.BlockSpec((1, _M, _C12), lambda b, i: (b, 0, 0)),
            pl.BlockSpec((_C11, _CH), lambda b, i: (0, 0)),
            pl.BlockSpec((_C12, _CH), lambda b, i: (0, 0)),
            pl.BlockSpec((1, _CH), lambda b, i: (0, 0)),
        ],
        out_specs=[
            pl.BlockSpec((1, _BN1, _CH), lambda b, i: (b, i, 0)),
            pl.BlockSpec((8, _CH), lambda b, i: (0, 0)),
        ],
        out_shape=[
            jax.ShapeDtypeStruct((_B, _N, _CH), jnp.float32),
            jax.ShapeDtypeStruct((8, _CH), jnp.float32),
        ],
        compiler_params=pltpu.CompilerParams(
            dimension_semantics=("arbitrary", "arbitrary")),
    )(dst_xyz, x2, dst_feat, src_xyzT, s2, src_feat, W1aT, W1bT, b1r)

    y1f = y1.reshape(_B * _N, _CH)
    nb2 = (_B * _N) // _BN2
    y2, stats2 = pl.pallas_call(
        _bn_relu_l2_kernel,
        grid=(nb2,),
        in_specs=[
            pl.BlockSpec((_BN2, _CH), lambda i: (i, 0)),
            pl.BlockSpec((8, _CH), lambda i: (0, 0)),
            pl.BlockSpec((1, _CH), lambda i: (0, 0)),
            pl.BlockSpec((1, _CH), lambda i: (0, 0)),
            pl.BlockSpec((_CH, _CH), lambda i: (0, 0)),
            pl.BlockSpec((1, _CH), lambda i: (0, 0)),
        ],
        out_specs=[
            pl.BlockSpec((_BN2, _CH), lambda i: (i, 0)),
            pl.BlockSpec((8, _CH), lambda i: (0, 0)),
        ],
        out_shape=[
            jax.ShapeDtypeStruct((_B * _N, _CH), jnp.float32),
            jax.ShapeDtypeStruct((8, _CH), jnp.float32),
        ],
        compiler_params=pltpu.CompilerParams(
            dimension_semantics=("arbitrary",)),
    )(y1f, stats1, g1r, be1r, W2T, b2r)

    out = pl.pallas_call(
        _bn_relu_kernel,
        grid=(nb2,),
        in_specs=[
            pl.BlockSpec((_BN2, _CH), lambda i: (i, 0)),
            pl.BlockSpec((8, _CH), lambda i: (0, 0)),
            pl.BlockSpec((1, _CH), lambda i: (0, 0)),
            pl.BlockSpec((1, _CH), lambda i: (0, 0)),
        ],
        out_specs=pl.BlockSpec((_BN2, _CH), lambda i: (i, 0)),
        out_shape=jax.ShapeDtypeStruct((_B * _N, _CH), jnp.float32),
        compiler_params=pltpu.CompilerParams(
            dimension_semantics=("arbitrary",)),
    )(y2, stats2, g2r, be2r)

    return out.reshape(_B, _N, _CH)


# speedup vs baseline: 21.2513x; 1.0835x over previous
import jax
import jax.numpy as jnp
from jax.experimental import pallas as pl
from jax.experimental.pallas import tpu as pltpu

_B, _N, _M = 4, 8192, 2048
_C11, _C12 = 32, 64
_CH = 64
_BN1 = 512
_BN2 = 2048
_NPTS = float(_B * _N)
_EPS_BN = 1e-5
_EPS_D = 1e-8


def _knn_interp_l1_kernel(dst_xyz_ref, x2_ref, dst_feat_ref, src_xyzT_ref,
                          s2_ref, src_feat_ref,
                          W1aT_ref, W1bT_ref, b1_ref, y1_ref, stats_ref):
    m = jnp.dot(dst_xyz_ref[0], src_xyzT_ref[0],
                preferred_element_type=jnp.float32)
    d = -2.0 * m
    d = d + x2_ref[0]
    d = d + s2_ref[0]

    iota = jax.lax.broadcasted_iota(jnp.int32, d.shape, 1)
    rs, hits = [], []
    dd = d
    for k in range(3):
        mval = jnp.min(dd, axis=1, keepdims=True)
        midx = jnp.min(jnp.where(dd == mval, iota, _M),
                       axis=1, keepdims=True)
        hit = iota == midx
        rs.append(1.0 / (mval + _EPS_D))
        hits.append(hit)
        if k < 2:
            dd = jnp.where(hit, jnp.float32(jnp.inf), dd)
    rsum = (rs[0] + rs[1]) + rs[2]
    wn = jnp.where(hits[0], rs[0] / rsum,
                   jnp.where(hits[1], rs[1] / rsum,
                             jnp.where(hits[2], rs[2] / rsum, 0.0)))

    interp = jnp.dot(wn, src_feat_ref[0],
                     preferred_element_type=jnp.float32,
                     precision=jax.lax.Precision.HIGHEST)
    y1 = (jnp.dot(dst_feat_ref[0], W1aT_ref[...],
                  preferred_element_type=jnp.float32)
          + jnp.dot(interp, W1bT_ref[...],
                    preferred_element_type=jnp.float32)
          + b1_ref[...])
    y1_ref[...] = y1[None]

    first = jnp.logical_and(pl.program_id(0) == 0, pl.program_id(1) == 0)

    @pl.when(first)
    def _init():
        stats_ref[...] = jnp.zeros_like(stats_ref)

    su = jnp.sum(y1, axis=0, keepdims=True)
    sq = jnp.sum(y1 * y1, axis=0, keepdims=True)
    stats_ref[...] += jnp.concatenate(
        [su, sq, jnp.zeros((6, _CH), jnp.float32)], axis=0)


def _bn_relu_l2_kernel(y1_ref, stats1_ref, g1_ref, be1_ref, W2T_ref, b2_ref,
                       y2_ref, stats2_ref):
    st = stats1_ref[...]
    mean = st[0:1, :] / _NPTS
    var = st[1:2, :] / _NPTS - mean * mean
    y = y1_ref[...]
    h = jnp.maximum(g1_ref[...] * (y - mean) * jax.lax.rsqrt(var + _EPS_BN)
                    + be1_ref[...], 0.0)
    y2 = jnp.dot(h, W2T_ref[...],
                 preferred_element_type=jnp.float32) + b2_ref[...]
    y2_ref[...] = y2

    @pl.when(pl.program_id(0) == 0)
    def _init():
        stats2_ref[...] = jnp.zeros_like(stats2_ref)

    su = jnp.sum(y2, axis=0, keepdims=True)
    sq = jnp.sum(y2 * y2, axis=0, keepdims=True)
    stats2_ref[...] += jnp.concatenate(
        [su, sq, jnp.zeros((6, _CH), jnp.float32)], axis=0)


def _bn_relu_kernel(y2_ref, stats2_ref, g2_ref, be2_ref, out_ref):
    st = stats2_ref[...]
    mean = st[0:1, :] / _NPTS
    var = st[1:2, :] / _NPTS - mean * mean
    y = y2_ref[...]
    out_ref[...] = jnp.maximum(
        g2_ref[...] * (y - mean) * jax.lax.rsqrt(var + _EPS_BN) + be2_ref[...],
        0.0)


def kernel(dst_xyz, dst_feat, src_xyz, src_feat, W1, b1, g1, be1, W2, b2, g2, be2):
    src_xyzT = jnp.transpose(src_xyz, (0, 2, 1))
    x2 = jnp.sum(dst_xyz ** 2, axis=-1, keepdims=True)
    s2 = jnp.sum(src_xyz ** 2, axis=-1)[:, None, :]
    W1aT = jnp.transpose(W1[:, :_C11])
    W1bT = jnp.transpose(W1[:, _C11:])
    W2T = jnp.transpose(W2)
    b1r = b1.reshape(1, _CH)
    b2r = b2.reshape(1, _CH)
    g1r = g1.reshape(1, _CH)
    be1r = be1.reshape(1, _CH)
    g2r = g2.reshape(1, _CH)
    be2r = be2.reshape(1, _CH)

    nb1 = _N // _BN1
    y1, stats1 = pl.pallas_call(
        _knn_interp_l1_kernel,
        grid=(_B, nb1),
        in_specs=[
            pl.BlockSpec((1, _BN1, 3), lambda b, i: (b, i, 0)),
            pl.BlockSpec((1, _BN1, 1), lambda b, i: (b, i, 0)),
            pl.BlockSpec((1, _BN1, _C11), lambda b, i: (b, i, 0)),
            pl.BlockSpec((1, 3, _M), lambda b, i: (b, 0, 0)),
            pl.BlockSpec((1, 1, _M), lambda b, i: (b, 0, 0)),
            pl.BlockSpec((1, _M, _C12), lambda b, i: (b, 0, 0)),
            pl.BlockSpec((_C11, _CH), lambda b, i: (0, 0)),
            pl.BlockSpec((_C12, _CH), lambda b, i: (0, 0)),
            pl.BlockSpec((1, _CH), lambda b, i: (0, 0)),
        ],
        out_specs=[
            pl.BlockSpec((1, _BN1, _CH), lambda b, i: (b, i, 0)),
            pl.BlockSpec((8, _CH), lambda b, i: (0, 0)),
        ],
        out_shape=[
            jax.ShapeDtypeStruct((_B, _N, _CH), jnp.float32),
            jax.ShapeDtypeStruct((8, _CH), jnp.float32),
        ],
        compiler_params=pltpu.CompilerParams(
            dimension_semantics=("arbitrary", "arbitrary")),
    )(dst_xyz, x2, dst_feat, src_xyzT, s2, src_feat, W1aT, W1bT, b1r)

    y1f = y1.reshape(_B * _N, _CH)
    nb2 = (_B * _N) // _BN2
    y2, stats2 = pl.pallas_call(
        _bn_relu_l2_kernel,
        grid=(nb2,),
        in_specs=[
            pl.BlockSpec((_BN2, _CH), lambda i: (i, 0)),
            pl.BlockSpec((8, _CH), lambda i: (0, 0)),
            pl.BlockSpec((1, _CH), lambda i: (0, 0)),
            pl.BlockSpec((1, _CH), lambda i: (0, 0)),
            pl.BlockSpec((_CH, _CH), lambda i: (0, 0)),
            pl.BlockSpec((1, _CH), lambda i: (0, 0)),
        ],
        out_specs=[
            pl.BlockSpec((_BN2, _CH), lambda i: (i, 0)),
            pl.BlockSpec((8, _CH), lambda i: (0, 0)),
        ],
        out_shape=[
            jax.ShapeDtypeStruct((_B * _N, _CH), jnp.float32),
            jax.ShapeDtypeStruct((8, _CH), jnp.float32),
        ],
        compiler_params=pltpu.CompilerParams(
            dimension_semantics=("arbitrary",)),
    )(y1f, stats1, g1r, be1r, W2T, b2r)

    out = pl.pallas_call(
        _bn_relu_kernel,
        grid=(nb2,),
        in_specs=[
            pl.BlockSpec((_BN2, _CH), lambda i: (i, 0)),
            pl.BlockSpec((8, _CH), lambda i: (0, 0)),
            pl.BlockSpec((1, _CH), lambda i: (0, 0)),
            pl.BlockSpec((1, _CH), lambda i: (0, 0)),
        ],
        out_specs=pl.BlockSpec((_BN2, _CH), lambda i: (i, 0)),
        out_shape=jax.ShapeDtypeStruct((_B * _N, _CH), jnp.float32),
        compiler_params=pltpu.CompilerParams(
            dimension_semantics=("arbitrary",)),
    )(y2, stats2, g2r, be2r)

    return out.reshape(_B, _N, _CH)


# speedup vs baseline: 29.9009x; 1.4070x over previous
import functools

import jax
import jax.numpy as jnp
from jax import lax
from jax.experimental import pallas as pl
from jax.experimental.pallas import tpu as pltpu
from jax.experimental.pallas import tpu_sc as plsc

_B, _N, _M = 4, 8192, 2048
_C11, _C12 = 32, 64
_CH = 64
_BN1 = 512
_BN2 = 2048
_NPTS = float(_B * _N)
_EPS_BN = 1e-5
_EPS_D = 1e-8

_NC, _NS = 2, 16
_NW = _NC * _NS
_NROWS = 3 * _B * _N
_RPW = _NROWS // _NW
_GCH = 128
_NGCH = _RPW // _GCH


def _knn_kernel(dst_xyz_ref, x2_ref, src_xyzT_ref, s2_ref, dk_ref, gidx_ref):
    m = jnp.dot(dst_xyz_ref[0], src_xyzT_ref[0],
                preferred_element_type=jnp.float32)
    d = -2.0 * m
    d = d + x2_ref[0]
    d = d + s2_ref[0]

    iota = jax.lax.broadcasted_iota(jnp.int32, d.shape, 1)
    vals, idxs = [], []
    dd = d
    for k in range(3):
        mval = jnp.min(dd, axis=1, keepdims=True)
        midx = jnp.min(jnp.where(dd == mval, iota, _M),
                       axis=1, keepdims=True)
        vals.append(mval)
        idxs.append(midx)
        if k < 2:
            dd = jnp.where(iota == midx, jnp.float32(jnp.inf), dd)
    dk_ref[...] = jnp.concatenate(vals, axis=1)[None]
    base = pl.program_id(0) * _M
    gidx_ref[...] = (jnp.concatenate(idxs, axis=1) + base)[None]


def _sc_gather_body(table_hbm, idx_hbm, out_hbm, idx_v, rows_v, sem):
    wid = lax.axis_index("s") * _NC + lax.axis_index("c")
    base = wid * _RPW

    def chunk(c, carry):
        off = base + c * _GCH
        pltpu.sync_copy(idx_hbm.at[pl.ds(off, _GCH)], idx_v)
        pltpu.async_copy(table_hbm.at[idx_v], rows_v, sem).wait()
        pltpu.sync_copy(rows_v, out_hbm.at[pl.ds(off, _GCH)])
        return carry

    lax.fori_loop(0, _NGCH, chunk, 0)


def _interp_l1_kernel(g0_ref, g1_ref, g2_ref, dk_ref, dst_feat_ref,
                      W1aT_ref, W1bT_ref, b1_ref, y1_ref, stats_ref):
    dk = dk_ref[...]
    r0 = 1.0 / (dk[:, 0:1] + _EPS_D)
    r1 = 1.0 / (dk[:, 1:2] + _EPS_D)
    r2 = 1.0 / (dk[:, 2:3] + _EPS_D)
    rsum = (r0 + r1) + r2
    interp = ((r0 / rsum) * g0_ref[...] + (r1 / rsum) * g1_ref[...]
              + (r2 / rsum) * g2_ref[...])
    y1 = (jnp.dot(dst_feat_ref[...], W1aT_ref[...],
                  preferred_element_type=jnp.float32)
          + jnp.dot(interp, W1bT_ref[...],
                    preferred_element_type=jnp.float32)
          + b1_ref[...])
    y1_ref[...] = y1

    @pl.when(pl.program_id(0) == 0)
    def _init():
        stats_ref[...] = jnp.zeros_like(stats_ref)

    su = jnp.sum(y1, axis=0, keepdims=True)
    sq = jnp.sum(y1 * y1, axis=0, keepdims=True)
    stats_ref[...] += jnp.concatenate(
        [su, sq, jnp.zeros((6, _CH), jnp.float32)], axis=0)


def _bn_relu_l2_kernel(y1_ref, stats1_ref, g1_ref, be1_ref, W2T_ref, b2_ref,
                       y2_ref, stats2_ref):
    st = stats1_ref[...]
    mean = st[0:1, :] / _NPTS
    var = st[1:2, :] / _NPTS - mean * mean
    y = y1_ref[...]
    h = jnp.maximum(g1_ref[...] * (y - mean) * jax.lax.rsqrt(var + _EPS_BN)
                    + be1_ref[...], 0.0)
    y2 = jnp.dot(h, W2T_ref[...],
                 preferred_element_type=jnp.float32) + b2_ref[...]
    y2_ref[...] = y2

    @pl.when(pl.program_id(0) == 0)
    def _init():
        stats2_ref[...] = jnp.zeros_like(stats2_ref)

    su = jnp.sum(y2, axis=0, keepdims=True)
    sq = jnp.sum(y2 * y2, axis=0, keepdims=True)
    stats2_ref[...] += jnp.concatenate(
        [su, sq, jnp.zeros((6, _CH), jnp.float32)], axis=0)


def _bn_relu_kernel(y2_ref, stats2_ref, g2_ref, be2_ref, out_ref):
    st = stats2_ref[...]
    mean = st[0:1, :] / _NPTS
    var = st[1:2, :] / _NPTS - mean * mean
    y = y2_ref[...]
    out_ref[...] = jnp.maximum(
        g2_ref[...] * (y - mean) * jax.lax.rsqrt(var + _EPS_BN) + be2_ref[...],
        0.0)


def kernel(dst_xyz, dst_feat, src_xyz, src_feat, W1, b1, g1, be1, W2, b2, g2, be2):
    src_xyzT = jnp.transpose(src_xyz, (0, 2, 1))
    x2 = jnp.sum(dst_xyz ** 2, axis=-1, keepdims=True)
    s2 = jnp.sum(src_xyz ** 2, axis=-1)[:, None, :]
    W1aT = jnp.transpose(W1[:, :_C11])
    W1bT = jnp.transpose(W1[:, _C11:])
    W2T = jnp.transpose(W2)
    b1r = b1.reshape(1, _CH)
    b2r = b2.reshape(1, _CH)
    g1r = g1.reshape(1, _CH)
    be1r = be1.reshape(1, _CH)
    g2r = g2.reshape(1, _CH)
    be2r = be2.reshape(1, _CH)

    nb1 = _N // _BN1
    dk, gidx = pl.pallas_call(
        _knn_kernel,
        grid=(_B, nb1),
        in_specs=[
            pl.BlockSpec((1, _BN1, 3), lambda b, i: (b, i, 0)),
            pl.BlockSpec((1, _BN1, 1), lambda b, i: (b, i, 0)),
            pl.BlockSpec((1, 3, _M), lambda b, i: (b, 0, 0)),
            pl.BlockSpec((1, 1, _M), lambda b, i: (b, 0, 0)),
        ],
        out_specs=[
            pl.BlockSpec((1, _BN1, 3), lambda b, i: (b, i, 0)),
            pl.BlockSpec((1, _BN1, 3), lambda b, i: (b, i, 0)),
        ],
        out_shape=[
            jax.ShapeDtypeStruct((_B, _N, 3), jnp.float32),
            jax.ShapeDtypeStruct((_B, _N, 3), jnp.int32),
        ],
        compiler_params=pltpu.CompilerParams(
            dimension_semantics=("arbitrary", "arbitrary")),
    )(dst_xyz, x2, src_xyzT, s2)

    gidx_km = jnp.transpose(gidx, (2, 0, 1)).reshape(_NROWS)
    table = src_feat.reshape(_B * _M, _C12)

    mesh = plsc.VectorSubcoreMesh(core_axis_name="c", subcore_axis_name="s")
    gath = pl.kernel(
        _sc_gather_body,
        out_type=jax.ShapeDtypeStruct((_NROWS, _C12), jnp.float32),
        mesh=mesh,
        scratch_types=[
            pltpu.VMEM((_GCH,), jnp.int32),
            pltpu.VMEM((_GCH, _C12), jnp.float32),
            pltpu.SemaphoreType.DMA,
        ],
        compiler_params=pltpu.CompilerParams(use_tc_tiling_on_sc=False),
    )(table, gidx_km)

    dkf = dk.reshape(_B * _N, 3)
    dff = dst_feat.reshape(_B * _N, _C11)
    nb2 = (_B * _N) // _BN2
    y1, stats1 = pl.pallas_call(
        _interp_l1_kernel,
        grid=(nb2,),
        in_specs=[
            pl.BlockSpec((_BN2, _C12), lambda i: (i, 0)),
            pl.BlockSpec((_BN2, _C12), lambda i, _nb=nb2: (i + _nb, 0)),
            pl.BlockSpec((_BN2, _C12), lambda i, _nb=nb2: (i + 2 * _nb, 0)),
            pl.BlockSpec((_BN2, 3), lambda i: (i, 0)),
            pl.BlockSpec((_BN2, _C11), lambda i: (i, 0)),
            pl.BlockSpec((_C11, _CH), lambda i: (0, 0)),
            pl.BlockSpec((_C12, _CH), lambda i: (0, 0)),
            pl.BlockSpec((1, _CH), lambda i: (0, 0)),
        ],
        out_specs=[
            pl.BlockSpec((_BN2, _CH), lambda i: (i, 0)),
            pl.BlockSpec((8, _CH), lambda i: (0, 0)),
        ],
        out_shape=[
            jax.ShapeDtypeStruct((_B * _N, _CH), jnp.float32),
            jax.ShapeDtypeStruct((8, _CH), jnp.float32),
        ],
        compiler_params=pltpu.CompilerParams(
            dimension_semantics=("arbitrary",)),
    )(gath, gath, gath, dkf, dff, W1aT, W1bT, b1r)

    y2, stats2 = pl.pallas_call(
        _bn_relu_l2_kernel,
        grid=(nb2,),
        in_specs=[
            pl.BlockSpec((_BN2, _CH), lambda i: (i, 0)),
            pl.BlockSpec((8, _CH), lambda i: (0, 0)),
            pl.BlockSpec((1, _CH), lambda i: (0, 0)),
            pl.BlockSpec((1, _CH), lambda i: (0, 0)),
            pl.BlockSpec((_CH, _CH), lambda i: (0, 0)),
            pl.BlockSpec((1, _CH), lambda i: (0, 0)),
        ],
        out_specs=[
            pl.BlockSpec((_BN2, _CH), lambda i: (i, 0)),
            pl.BlockSpec((8, _CH), lambda i: (0, 0)),
        ],
        out_shape=[
            jax.ShapeDtypeStruct((_B * _N, _CH), jnp.float32),
            jax.ShapeDtypeStruct((8, _CH), jnp.float32),
        ],
        compiler_params=pltpu.CompilerParams(
            dimension_semantics=("arbitrary",)),
    )(y1, stats1, g1r, be1r, W2T, b2r)

    out = pl.pallas_call(
        _bn_relu_kernel,
        grid=(nb2,),
        in_specs=[
            pl.BlockSpec((_BN2, _CH), lambda i: (i, 0)),
            pl.BlockSpec((8, _CH), lambda i: (0, 0)),
            pl.BlockSpec((1, _CH), lambda i: (0, 0)),
            pl.BlockSpec((1, _CH), lambda i: (0, 0)),
        ],
        out_specs=pl.BlockSpec((_BN2, _CH), lambda i: (i, 0)),
        out_shape=jax.ShapeDtypeStruct((_B * _N, _CH), jnp.float32),
        compiler_params=pltpu.CompilerParams(
            dimension_semantics=("arbitrary",)),
    )(y2, stats2, g2r, be2r)

    return out.reshape(_B, _N, _CH)


# speedup vs baseline: 31.4744x; 1.0526x over previous
import functools

import jax
import jax.numpy as jnp
from jax import lax
from jax.experimental import pallas as pl
from jax.experimental.pallas import tpu as pltpu
from jax.experimental.pallas import tpu_sc as plsc

_B, _N, _M = 4, 8192, 2048
_C11, _C12 = 32, 64
_CH = 64
_BN1 = 512
_BN2 = 2048
_NPTS = float(_B * _N)
_EPS_BN = 1e-5
_EPS_D = 1e-8

_NC, _NS = 2, 16
_NW = _NC * _NS
_NROWS = 3 * _B * _N
_RPW = _NROWS // _NW
_GCH = 128
_NGCH = _RPW // _GCH


def _knn_kernel(xm2_ref, x2_ref, src_xyzT_ref, s2_ref, dk_ref, gidx_ref):
    d = jnp.dot(xm2_ref[0], src_xyzT_ref[0],
                preferred_element_type=jnp.float32)
    d = d + x2_ref[0]
    d = d + s2_ref[0]

    iota_i = jax.lax.broadcasted_iota(jnp.int32, d.shape, 1)
    key = jax.lax.bitcast_convert_type(iota_i | jnp.int32(0x3F800000),
                                       jnp.float32)
    vals, idxs = [], []
    dd = d
    for k in range(3):
        mval = jnp.min(dd, axis=1, keepdims=True)
        mkey = jnp.min(jnp.where(dd == mval, key, jnp.float32(2.0)),
                       axis=1, keepdims=True)
        vals.append(mval)
        idxs.append(jax.lax.bitcast_convert_type(mkey, jnp.int32)
                    & jnp.int32(0x007FFFFF))
        if k < 2:
            dd = jnp.where(key == mkey, jnp.float32(jnp.inf), dd)
    dk_ref[...] = jnp.concatenate(vals, axis=1)[None]
    base = pl.program_id(0) * _M
    gidx_ref[...] = (jnp.concatenate(idxs, axis=1) + base)[None]


def _sc_gather_body(table_hbm, idx_hbm, out_hbm, idx_v, rows_v, sem):
    wid = lax.axis_index("s") * _NC + lax.axis_index("c")
    base = wid * _RPW

    def chunk(c, carry):
        off = base + c * _GCH
        pltpu.sync_copy(idx_hbm.at[pl.ds(off, _GCH)], idx_v)
        pltpu.async_copy(table_hbm.at[idx_v], rows_v, sem).wait()
        pltpu.sync_copy(rows_v, out_hbm.at[pl.ds(off, _GCH)])
        return carry

    lax.fori_loop(0, _NGCH, chunk, 0)


def _interp_l1_kernel(g0_ref, g1_ref, g2_ref, dk_ref, dst_feat_ref,
                      W1aT_ref, W1bT_ref, b1_ref, y1_ref, stats_ref):
    r = 1.0 / (dk_ref[...] + _EPS_D)
    r0, r1, r2 = r[:, 0:1], r[:, 1:2], r[:, 2:3]
    inv = 1.0 / ((r0 + r1) + r2)
    interp = ((r0 * inv) * g0_ref[...] + (r1 * inv) * g1_ref[...]
              + (r2 * inv) * g2_ref[...])
    y1 = (jnp.dot(dst_feat_ref[...], W1aT_ref[...],
                  preferred_element_type=jnp.float32)
          + jnp.dot(interp, W1bT_ref[...],
                    preferred_element_type=jnp.float32)
          + b1_ref[...])
    y1_ref[...] = y1

    @pl.when(pl.program_id(0) == 0)
    def _init():
        stats_ref[...] = jnp.zeros_like(stats_ref)

    su = jnp.sum(y1, axis=0, keepdims=True)
    sq = jnp.sum(y1 * y1, axis=0, keepdims=True)
    stats_ref[...] += jnp.concatenate(
        [su, sq, jnp.zeros((6, _CH), jnp.float32)], axis=0)


def _bn_relu_l2_kernel(y1_ref, stats1_ref, g1_ref, be1_ref, W2T_ref, b2_ref,
                       y2_ref, stats2_ref):
    st = stats1_ref[...]
    mean = st[0:1, :] / _NPTS
    var = st[1:2, :] / _NPTS - mean * mean
    y = y1_ref[...]
    h = jnp.maximum(g1_ref[...] * (y - mean) * jax.lax.rsqrt(var + _EPS_BN)
                    + be1_ref[...], 0.0)
    y2 = jnp.dot(h, W2T_ref[...],
                 preferred_element_type=jnp.float32) + b2_ref[...]
    y2_ref[...] = y2

    @pl.when(pl.program_id(0) == 0)
    def _init():
        stats2_ref[...] = jnp.zeros_like(stats2_ref)

    su = jnp.sum(y2, axis=0, keepdims=True)
    sq = jnp.sum(y2 * y2, axis=0, keepdims=True)
    stats2_ref[...] += jnp.concatenate(
        [su, sq, jnp.zeros((6, _CH), jnp.float32)], axis=0)


def _bn_relu_kernel(y2_ref, stats2_ref, g2_ref, be2_ref, out_ref):
    st = stats2_ref[...]
    mean = st[0:1, :] / _NPTS
    var = st[1:2, :] / _NPTS - mean * mean
    y = y2_ref[...]
    out_ref[...] = jnp.maximum(
        g2_ref[...] * (y - mean) * jax.lax.rsqrt(var + _EPS_BN) + be2_ref[...],
        0.0)


def kernel(dst_xyz, dst_feat, src_xyz, src_feat, W1, b1, g1, be1, W2, b2, g2, be2):
    src_xyzT = jnp.transpose(src_xyz, (0, 2, 1))
    xm2 = -2.0 * dst_xyz
    x2 = jnp.sum(dst_xyz ** 2, axis=-1, keepdims=True)
    s2 = jnp.sum(src_xyz ** 2, axis=-1)[:, None, :]
    W1aT = jnp.transpose(W1[:, :_C11])
    W1bT = jnp.transpose(W1[:, _C11:])
    W2T = jnp.transpose(W2)
    b1r = b1.reshape(1, _CH)
    b2r = b2.reshape(1, _CH)
    g1r = g1.reshape(1, _CH)
    be1r = be1.reshape(1, _CH)
    g2r = g2.reshape(1, _CH)
    be2r = be2.reshape(1, _CH)

    nb1 = _N // _BN1
    dk, gidx = pl.pallas_call(
        _knn_kernel,
        grid=(_B, nb1),
        in_specs=[
            pl.BlockSpec((1, _BN1, 3), lambda b, i: (b, i, 0)),
            pl.BlockSpec((1, _BN1, 1), lambda b, i: (b, i, 0)),
            pl.BlockSpec((1, 3, _M), lambda b, i: (b, 0, 0)),
            pl.BlockSpec((1, 1, _M), lambda b, i: (b, 0, 0)),
        ],
        out_specs=[
            pl.BlockSpec((1, _BN1, 3), lambda b, i: (b, i, 0)),
            pl.BlockSpec((1, _BN1, 3), lambda b, i: (b, i, 0)),
        ],
        out_shape=[
            jax.ShapeDtypeStruct((_B, _N, 3), jnp.float32),
            jax.ShapeDtypeStruct((_B, _N, 3), jnp.int32),
        ],
        compiler_params=pltpu.CompilerParams(
            dimension_semantics=("arbitrary", "arbitrary")),
    )(xm2, x2, src_xyzT, s2)

    gidx_km = jnp.transpose(gidx, (2, 0, 1)).reshape(_NROWS)
    table = src_feat.reshape(_B * _M, _C12)

    mesh = plsc.VectorSubcoreMesh(core_axis_name="c", subcore_axis_name="s")
    gath = pl.kernel(
        _sc_gather_body,
        out_type=jax.ShapeDtypeStruct((_NROWS, _C12), jnp.float32),
        mesh=mesh,
        scratch_types=[
            pltpu.VMEM((_GCH,), jnp.int32),
            pltpu.VMEM((_GCH, _C12), jnp.float32),
            pltpu.SemaphoreType.DMA,
        ],
        compiler_params=pltpu.CompilerParams(use_tc_tiling_on_sc=False),
    )(table, gidx_km)

    dkf = dk.reshape(_B * _N, 3)
    dff = dst_feat.reshape(_B * _N, _C11)
    nb2 = (_B * _N) // _BN2
    y1, stats1 = pl.pallas_call(
        _interp_l1_kernel,
        grid=(nb2,),
        in_specs=[
            pl.BlockSpec((_BN2, _C12), lambda i: (i, 0)),
            pl.BlockSpec((_BN2, _C12), lambda i, _nb=nb2: (i + _nb, 0)),
            pl.BlockSpec((_BN2, _C12), lambda i, _nb=nb2: (i + 2 * _nb, 0)),
            pl.BlockSpec((_BN2, 3), lambda i: (i, 0)),
            pl.BlockSpec((_BN2, _C11), lambda i: (i, 0)),
            pl.BlockSpec((_C11, _CH), lambda i: (0, 0)),
            pl.BlockSpec((_C12, _CH), lambda i: (0, 0)),
            pl.BlockSpec((1, _CH), lambda i: (0, 0)),
        ],
        out_specs=[
            pl.BlockSpec((_BN2, _CH), lambda i: (i, 0)),
            pl.BlockSpec((8, _CH), lambda i: (0, 0)),
        ],
        out_shape=[
            jax.ShapeDtypeStruct((_B * _N, _CH), jnp.float32),
            jax.ShapeDtypeStruct((8, _CH), jnp.float32),
        ],
        compiler_params=pltpu.CompilerParams(
            dimension_semantics=("arbitrary",)),
    )(gath, gath, gath, dkf, dff, W1aT, W1bT, b1r)

    y2, stats2 = pl.pallas_call(
        _bn_relu_l2_kernel,
        grid=(nb2,),
        in_specs=[
            pl.BlockSpec((_BN2, _CH), lambda i: (i, 0)),
            pl.BlockSpec((8, _CH), lambda i: (0, 0)),
            pl.BlockSpec((1, _CH), lambda i: (0, 0)),
            pl.BlockSpec((1, _CH), lambda i: (0, 0)),
            pl.BlockSpec((_CH, _CH), lambda i: (0, 0)),
            pl.BlockSpec((1, _CH), lambda i: (0, 0)),
        ],
        out_specs=[
            pl.BlockSpec((_BN2, _CH), lambda i: (i, 0)),
            pl.BlockSpec((8, _CH), lambda i: (0, 0)),
        ],
        out_shape=[
            jax.ShapeDtypeStruct((_B * _N, _CH), jnp.float32),
            jax.ShapeDtypeStruct((8, _CH), jnp.float32),
        ],
        compiler_params=pltpu.CompilerParams(
            dimension_semantics=("arbitrary",)),
    )(y1, stats1, g1r, be1r, W2T, b2r)

    out = pl.pallas_call(
        _bn_relu_kernel,
        grid=(nb2,),
        in_specs=[
            pl.BlockSpec((_BN2, _CH), lambda i: (i, 0)),
            pl.BlockSpec((8, _CH), lambda i: (0, 0)),
            pl.BlockSpec((1, _CH), lambda i: (0, 0)),
            pl.BlockSpec((1, _CH), lambda i: (0, 0)),
        ],
        out_specs=pl.BlockSpec((_BN2, _CH), lambda i: (i, 0)),
        out_shape=jax.ShapeDtypeStruct((_B * _N, _CH), jnp.float32),
        compiler_params=pltpu.CompilerParams(
            dimension_semantics=("arbitrary",)),
    )(y2, stats2, g2r, be2r)

    return out.reshape(_B, _N, _CH)


# speedup vs baseline: 31.8716x; 1.0126x over previous
import functools

import jax
import jax.numpy as jnp
from jax import lax
from jax.experimental import pallas as pl
from jax.experimental.pallas import tpu as pltpu
from jax.experimental.pallas import tpu_sc as plsc

_B, _N, _M = 4, 8192, 2048
_C11, _C12 = 32, 64
_CH = 64
_BN1 = 1024
_BN2 = 2048
_NPTS = float(_B * _N)
_EPS_BN = 1e-5
_EPS_D = 1e-8

_NC, _NS = 2, 16
_NW = _NC * _NS
_NROWS = 3 * _B * _N
_RPW = _NROWS // _NW
_GCH = 128
_NGCH = _RPW // _GCH


def _knn_kernel(xm2_ref, x2_ref, src_xyzT_ref, s2_ref, dk_ref, gidx_ref):
    d = jnp.dot(xm2_ref[0], src_xyzT_ref[0],
                preferred_element_type=jnp.float32)
    d = d + x2_ref[0]
    d = d + s2_ref[0]

    iota_i = jax.lax.broadcasted_iota(jnp.int32, d.shape, 1)
    key = jax.lax.bitcast_convert_type(iota_i | jnp.int32(0x3F800000),
                                       jnp.float32)
    vals, idxs = [], []
    dd = d
    for k in range(3):
        mval = jnp.min(dd, axis=1, keepdims=True)
        mkey = jnp.min(jnp.where(dd == mval, key, jnp.float32(2.0)),
                       axis=1, keepdims=True)
        vals.append(mval)
        idxs.append(jax.lax.bitcast_convert_type(mkey, jnp.int32)
                    & jnp.int32(0x007FFFFF))
        if k < 2:
            dd = jnp.where(key == mkey, jnp.float32(jnp.inf), dd)
    dk_ref[...] = jnp.concatenate(vals, axis=1)[None]
    base = pl.program_id(0) * _M
    gidx_ref[...] = (jnp.concatenate(idxs, axis=1) + base)[None]


def _sc_gather_body(table_hbm, idx_hbm, out_hbm, idx_v, rows_v, sem):
    wid = lax.axis_index("s") * _NC + lax.axis_index("c")
    base = wid * _RPW

    def chunk(c, carry):
        off = base + c * _GCH
        pltpu.sync_copy(idx_hbm.at[pl.ds(off, _GCH)], idx_v)
        pltpu.async_copy(table_hbm.at[idx_v], rows_v, sem).wait()
        pltpu.sync_copy(rows_v, out_hbm.at[pl.ds(off, _GCH)])
        return carry

    lax.fori_loop(0, _NGCH, chunk, 0)


def _interp_l1_kernel(g0_ref, g1_ref, g2_ref, dk_ref, dst_feat_ref,
                      W1aT_ref, W1bT_ref, b1_ref, y1_ref, stats_ref):
    r = 1.0 / (dk_ref[...] + _EPS_D)
    r0, r1, r2 = r[:, 0:1], r[:, 1:2], r[:, 2:3]
    inv = 1.0 / ((r0 + r1) + r2)
    interp = ((r0 * inv) * g0_ref[...] + (r1 * inv) * g1_ref[...]
              + (r2 * inv) * g2_ref[...])
    y1 = (jnp.dot(dst_feat_ref[...], W1aT_ref[...],
                  preferred_element_type=jnp.float32)
          + jnp.dot(interp, W1bT_ref[...],
                    preferred_element_type=jnp.float32)
          + b1_ref[...])
    y1_ref[...] = y1

    @pl.when(pl.program_id(0) == 0)
    def _init():
        stats_ref[...] = jnp.zeros_like(stats_ref)

    su = jnp.sum(y1, axis=0, keepdims=True)
    sq = jnp.sum(y1 * y1, axis=0, keepdims=True)
    stats_ref[...] += jnp.concatenate(
        [su, sq, jnp.zeros((6, _CH), jnp.float32)], axis=0)


def _bn_relu_l2_kernel(y1_ref, stats1_ref, g1_ref, be1_ref, W2T_ref, b2_ref,
                       y2_ref, stats2_ref):
    st = stats1_ref[...]
    mean = st[0:1, :] / _NPTS
    var = st[1:2, :] / _NPTS - mean * mean
    y = y1_ref[...]
    h = jnp.maximum(g1_ref[...] * (y - mean) * jax.lax.rsqrt(var + _EPS_BN)
                    + be1_ref[...], 0.0)
    y2 = jnp.dot(h, W2T_ref[...],
                 preferred_element_type=jnp.float32) + b2_ref[...]
    y2_ref[...] = y2

    @pl.when(pl.program_id(0) == 0)
    def _init():
        stats2_ref[...] = jnp.zeros_like(stats2_ref)

    su = jnp.sum(y2, axis=0, keepdims=True)
    sq = jnp.sum(y2 * y2, axis=0, keepdims=True)
    stats2_ref[...] += jnp.concatenate(
        [su, sq, jnp.zeros((6, _CH), jnp.float32)], axis=0)


def _bn_relu_kernel(y2_ref, stats2_ref, g2_ref, be2_ref, out_ref):
    st = stats2_ref[...]
    mean = st[0:1, :] / _NPTS
    var = st[1:2, :] / _NPTS - mean * mean
    y = y2_ref[...]
    out_ref[...] = jnp.maximum(
        g2_ref[...] * (y - mean) * jax.lax.rsqrt(var + _EPS_BN) + be2_ref[...],
        0.0)


def kernel(dst_xyz, dst_feat, src_xyz, src_feat, W1, b1, g1, be1, W2, b2, g2, be2):
    src_xyzT = jnp.transpose(src_xyz, (0, 2, 1))
    xm2 = -2.0 * dst_xyz
    x2 = jnp.sum(dst_xyz ** 2, axis=-1, keepdims=True)
    s2 = jnp.sum(src_xyz ** 2, axis=-1)[:, None, :]
    W1aT = jnp.transpose(W1[:, :_C11])
    W1bT = jnp.transpose(W1[:, _C11:])
    W2T = jnp.transpose(W2)
    b1r = b1.reshape(1, _CH)
    b2r = b2.reshape(1, _CH)
    g1r = g1.reshape(1, _CH)
    be1r = be1.reshape(1, _CH)
    g2r = g2.reshape(1, _CH)
    be2r = be2.reshape(1, _CH)

    nb1 = _N // _BN1
    dk, gidx = pl.pallas_call(
        _knn_kernel,
        grid=(_B, nb1),
        in_specs=[
            pl.BlockSpec((1, _BN1, 3), lambda b, i: (b, i, 0)),
            pl.BlockSpec((1, _BN1, 1), lambda b, i: (b, i, 0)),
            pl.BlockSpec((1, 3, _M), lambda b, i: (b, 0, 0)),
            pl.BlockSpec((1, 1, _M), lambda b, i: (b, 0, 0)),
        ],
        out_specs=[
            pl.BlockSpec((1, _BN1, 3), lambda b, i: (b, i, 0)),
            pl.BlockSpec((1, _BN1, 3), lambda b, i: (b, i, 0)),
        ],
        out_shape=[
            jax.ShapeDtypeStruct((_B, _N, 3), jnp.float32),
            jax.ShapeDtypeStruct((_B, _N, 3), jnp.int32),
        ],
        compiler_params=pltpu.CompilerParams(
            dimension_semantics=("arbitrary", "arbitrary")),
    )(xm2, x2, src_xyzT, s2)

    gidx_km = jnp.transpose(gidx, (2, 0, 1)).reshape(_NROWS)
    table = src_feat.reshape(_B * _M, _C12)

    mesh = plsc.VectorSubcoreMesh(core_axis_name="c", subcore_axis_name="s")
    gath = pl.kernel(
        _sc_gather_body,
        out_type=jax.ShapeDtypeStruct((_NROWS, _C12), jnp.float32),
        mesh=mesh,
        scratch_types=[
            pltpu.VMEM((_GCH,), jnp.int32),
            pltpu.VMEM((_GCH, _C12), jnp.float32),
            pltpu.SemaphoreType.DMA,
        ],
        compiler_params=pltpu.CompilerParams(use_tc_tiling_on_sc=False),
    )(table, gidx_km)

    dkf = dk.reshape(_B * _N, 3)
    dff = dst_feat.reshape(_B * _N, _C11)
    nb2 = (_B * _N) // _BN2
    y1, stats1 = pl.pallas_call(
        _interp_l1_kernel,
        grid=(nb2,),
        in_specs=[
            pl.BlockSpec((_BN2, _C12), lambda i: (i, 0)),
            pl.BlockSpec((_BN2, _C12), lambda i, _nb=nb2: (i + _nb, 0)),
            pl.BlockSpec((_BN2, _C12), lambda i, _nb=nb2: (i + 2 * _nb, 0)),
            pl.BlockSpec((_BN2, 3), lambda i: (i, 0)),
            pl.BlockSpec((_BN2, _C11), lambda i: (i, 0)),
            pl.BlockSpec((_C11, _CH), lambda i: (0, 0)),
            pl.BlockSpec((_C12, _CH), lambda i: (0, 0)),
            pl.BlockSpec((1, _CH), lambda i: (0, 0)),
        ],
        out_specs=[
            pl.BlockSpec((_BN2, _CH), lambda i: (i, 0)),
            pl.BlockSpec((8, _CH), lambda i: (0, 0)),
        ],
        out_shape=[
            jax.ShapeDtypeStruct((_B * _N, _CH), jnp.float32),
            jax.ShapeDtypeStruct((8, _CH), jnp.float32),
        ],
        compiler_params=pltpu.CompilerParams(
            dimension_semantics=("arbitrary",)),
    )(gath, gath, gath, dkf, dff, W1aT, W1bT, b1r)

    y2, stats2 = pl.pallas_call(
        _bn_relu_l2_kernel,
        grid=(nb2,),
        in_specs=[
            pl.BlockSpec((_BN2, _CH), lambda i: (i, 0)),
            pl.BlockSpec((8, _CH), lambda i: (0, 0)),
            pl.BlockSpec((1, _CH), lambda i: (0, 0)),
            pl.BlockSpec((1, _CH), lambda i: (0, 0)),
            pl.BlockSpec((_CH, _CH), lambda i: (0, 0)),
            pl.BlockSpec((1, _CH), lambda i: (0, 0)),
        ],
        out_specs=[
            pl.BlockSpec((_BN2, _CH), lambda i: (i, 0)),
            pl.BlockSpec((8, _CH), lambda i: (0, 0)),
        ],
        out_shape=[
            jax.ShapeDtypeStruct((_B * _N, _CH), jnp.float32),
            jax.ShapeDtypeStruct((8, _CH), jnp.float32),
        ],
        compiler_params=pltpu.CompilerParams(
            dimension_semantics=("arbitrary",)),
    )(y1, stats1, g1r, be1r, W2T, b2r)

    out = pl.pallas_call(
        _bn_relu_kernel,
        grid=(nb2,),
        in_specs=[
            pl.BlockSpec((_BN2, _CH), lambda i: (i, 0)),
            pl.BlockSpec((8, _CH), lambda i: (0, 0)),
            pl.BlockSpec((1, _CH), lambda i: (0, 0)),
            pl.BlockSpec((1, _CH), lambda i: (0, 0)),
        ],
        out_specs=pl.BlockSpec((_BN2, _CH), lambda i: (i, 0)),
        out_shape=jax.ShapeDtypeStruct((_B * _N, _CH), jnp.float32),
        compiler_params=pltpu.CompilerParams(
            dimension_semantics=("arbitrary",)),
    )(y2, stats2, g2r, be2r)

    return out.reshape(_B, _N, _CH)


# speedup vs baseline: 33.2613x; 1.0436x over previous
import functools

import jax
import jax.numpy as jnp
from jax import lax
from jax.experimental import pallas as pl
from jax.experimental.pallas import tpu as pltpu
from jax.experimental.pallas import tpu_sc as plsc

_B, _N, _M = 4, 8192, 2048
_C11, _C12 = 32, 64
_CH = 64
_BN1 = 1024
_BN2 = 2048
_NPTS = float(_B * _N)
_EPS_BN = 1e-5
_EPS_D = 1e-8

_NC, _NS = 2, 16
_NW = _NC * _NS
_NROWS = 3 * _B * _N
_RPW = _NROWS // _NW
_GCH = 128
_NGCH = _RPW // _GCH


def _knn_kernel(xm2_ref, x2_ref, src_xyzT_ref, s2_ref, dk_ref, gidx_ref):
    d = jnp.dot(xm2_ref[0], src_xyzT_ref[0],
                preferred_element_type=jnp.float32)
    d = d + x2_ref[0]
    d = d + s2_ref[0]

    iota_i = jax.lax.broadcasted_iota(jnp.int32, d.shape, 1)
    key = jax.lax.bitcast_convert_type(iota_i | jnp.int32(0x3F800000),
                                       jnp.float32)
    vals, idxs = [], []
    dd = d
    for k in range(3):
        mval = jnp.min(dd, axis=1, keepdims=True)
        mkey = jnp.min(jnp.where(dd == mval, key, jnp.float32(2.0)),
                       axis=1, keepdims=True)
        vals.append(mval)
        idxs.append(jax.lax.bitcast_convert_type(mkey, jnp.int32)
                    & jnp.int32(0x007FFFFF))
        if k < 2:
            dd = jnp.where(key == mkey, jnp.float32(jnp.inf), dd)
    dk_ref[...] = jnp.concatenate(vals, axis=1)[None]
    base = pl.program_id(0) * _M
    gidx_ref[...] = (jnp.concatenate(idxs, axis=1) + base)[None]


def _sc_gather_body(table_hbm, idx_hbm, out_hbm, idx_v, rows_v, sem):
    wid = lax.axis_index("s") * _NC + lax.axis_index("c")
    base = wid * _RPW

    def chunk(c, carry):
        off = base + c * _GCH
        pltpu.sync_copy(idx_hbm.at[pl.ds(off, _GCH)], idx_v)
        pltpu.async_copy(table_hbm.at[idx_v], rows_v, sem).wait()
        pltpu.sync_copy(rows_v, out_hbm.at[pl.ds(off, _GCH)])
        return carry

    lax.fori_loop(0, _NGCH, chunk, 0)


def _mlp_fused_kernel(g0_ref, g1_ref, g2_ref, dk_ref, dst_feat_ref,
                      W1aT_ref, W1bT_ref, b1_ref, g1p_ref, be1p_ref,
                      W2T_ref, b2_ref, g2p_ref, be2p_ref,
                      out_ref, y1s, y2s, st1, st2):
    p = pl.program_id(0)
    i = pl.program_id(1)
    rows = pl.ds(i * _BN2, _BN2)

    @pl.when(p == 0)
    def _phase0():
        r = 1.0 / (dk_ref[...] + _EPS_D)
        r0, r1, r2 = r[:, 0:1], r[:, 1:2], r[:, 2:3]
        inv = 1.0 / ((r0 + r1) + r2)
        interp = ((r0 * inv) * g0_ref[...] + (r1 * inv) * g1_ref[...]
                  + (r2 * inv) * g2_ref[...])
        y1 = (jnp.dot(dst_feat_ref[...], W1aT_ref[...],
                      preferred_element_type=jnp.float32)
              + jnp.dot(interp, W1bT_ref[...],
                        preferred_element_type=jnp.float32)
              + b1_ref[...])
        y1s[rows, :] = y1

        @pl.when(i == 0)
        def _init():
            st1[...] = jnp.zeros_like(st1)

        su = jnp.sum(y1, axis=0, keepdims=True)
        sq = jnp.sum(y1 * y1, axis=0, keepdims=True)
        st1[...] += jnp.concatenate(
            [su, sq, jnp.zeros((6, _CH), jnp.float32)], axis=0)

    @pl.when(p == 1)
    def _phase1():
        st = st1[...]
        mean = st[0:1, :] / _NPTS
        var = st[1:2, :] / _NPTS - mean * mean
        y = y1s[rows, :]
        h = jnp.maximum(
            g1p_ref[...] * (y - mean) * jax.lax.rsqrt(var + _EPS_BN)
            + be1p_ref[...], 0.0)
        y2 = jnp.dot(h, W2T_ref[...],
                     preferred_element_type=jnp.float32) + b2_ref[...]
        y2s[rows, :] = y2

        @pl.when(i == 0)
        def _init():
            st2[...] = jnp.zeros_like(st2)

        su = jnp.sum(y2, axis=0, keepdims=True)
        sq = jnp.sum(y2 * y2, axis=0, keepdims=True)
        st2[...] += jnp.concatenate(
            [su, sq, jnp.zeros((6, _CH), jnp.float32)], axis=0)

    @pl.when(p == 2)
    def _phase2():
        st = st2[...]
        mean = st[0:1, :] / _NPTS
        var = st[1:2, :] / _NPTS - mean * mean
        y = y2s[rows, :]
        out_ref[...] = jnp.maximum(
            g2p_ref[...] * (y - mean) * jax.lax.rsqrt(var + _EPS_BN)
            + be2p_ref[...], 0.0)


def kernel(dst_xyz, dst_feat, src_xyz, src_feat, W1, b1, g1, be1, W2, b2, g2, be2):
    src_xyzT = jnp.transpose(src_xyz, (0, 2, 1))
    xm2 = -2.0 * dst_xyz
    x2 = jnp.sum(dst_xyz ** 2, axis=-1, keepdims=True)
    s2 = jnp.sum(src_xyz ** 2, axis=-1)[:, None, :]
    W1aT = jnp.transpose(W1[:, :_C11])
    W1bT = jnp.transpose(W1[:, _C11:])
    W2T = jnp.transpose(W2)
    b1r = b1.reshape(1, _CH)
    b2r = b2.reshape(1, _CH)
    g1r = g1.reshape(1, _CH)
    be1r = be1.reshape(1, _CH)
    g2r = g2.reshape(1, _CH)
    be2r = be2.reshape(1, _CH)

    nb1 = _N // _BN1
    dk, gidx = pl.pallas_call(
        _knn_kernel,
        grid=(_B, nb1),
        in_specs=[
            pl.BlockSpec((1, _BN1, 3), lambda b, i: (b, i, 0)),
            pl.BlockSpec((1, _BN1, 1), lambda b, i: (b, i, 0)),
            pl.BlockSpec((1, 3, _M), lambda b, i: (b, 0, 0)),
            pl.BlockSpec((1, 1, _M), lambda b, i: (b, 0, 0)),
        ],
        out_specs=[
            pl.BlockSpec((1, _BN1, 3), lambda b, i: (b, i, 0)),
            pl.BlockSpec((1, _BN1, 3), lambda b, i: (b, i, 0)),
        ],
        out_shape=[
            jax.ShapeDtypeStruct((_B, _N, 3), jnp.float32),
            jax.ShapeDtypeStruct((_B, _N, 3), jnp.int32),
        ],
        compiler_params=pltpu.CompilerParams(
            dimension_semantics=("arbitrary", "arbitrary")),
    )(xm2, x2, src_xyzT, s2)

    gidx_km = jnp.transpose(gidx, (2, 0, 1)).reshape(_NROWS)
    table = src_feat.reshape(_B * _M, _C12)

    mesh = plsc.VectorSubcoreMesh(core_axis_name="c", subcore_axis_name="s")
    gath = pl.kernel(
        _sc_gather_body,
        out_type=jax.ShapeDtypeStruct((_NROWS, _C12), jnp.float32),
        mesh=mesh,
        scratch_types=[
            pltpu.VMEM((_GCH,), jnp.int32),
            pltpu.VMEM((_GCH, _C12), jnp.float32),
            pltpu.SemaphoreType.DMA,
        ],
        compiler_params=pltpu.CompilerParams(use_tc_tiling_on_sc=False),
    )(table, gidx_km)

    dkf = dk.reshape(_B * _N, 3)
    dff = dst_feat.reshape(_B * _N, _C11)
    nb2 = (_B * _N) // _BN2

    def _ph0_map(p, i, off=0):
        return (jnp.where(p == 0, i + off, 0), 0)

    out = pl.pallas_call(
        _mlp_fused_kernel,
        grid=(3, nb2),
        in_specs=[
            pl.BlockSpec((_BN2, _C12), functools.partial(_ph0_map, off=0)),
            pl.BlockSpec((_BN2, _C12), functools.partial(_ph0_map, off=nb2)),
            pl.BlockSpec((_BN2, _C12),
                         functools.partial(_ph0_map, off=2 * nb2)),
            pl.BlockSpec((_BN2, 3), _ph0_map),
            pl.BlockSpec((_BN2, _C11), _ph0_map),
            pl.BlockSpec((_C11, _CH), lambda p, i: (0, 0)),
            pl.BlockSpec((_C12, _CH), lambda p, i: (0, 0)),
            pl.BlockSpec((1, _CH), lambda p, i: (0, 0)),
            pl.BlockSpec((1, _CH), lambda p, i: (0, 0)),
            pl.BlockSpec((1, _CH), lambda p, i: (0, 0)),
            pl.BlockSpec((_CH, _CH), lambda p, i: (0, 0)),
            pl.BlockSpec((1, _CH), lambda p, i: (0, 0)),
            pl.BlockSpec((1, _CH), lambda p, i: (0, 0)),
            pl.BlockSpec((1, _CH), lambda p, i: (0, 0)),
        ],
        out_specs=pl.BlockSpec((_BN2, _CH),
                               lambda p, i: (jnp.where(p == 2, i, 0), 0)),
        out_shape=jax.ShapeDtypeStruct((_B * _N, _CH), jnp.float32),
        scratch_shapes=[
            pltpu.VMEM((_B * _N, _CH), jnp.float32),
            pltpu.VMEM((_B * _N, _CH), jnp.float32),
            pltpu.VMEM((8, _CH), jnp.float32),
            pltpu.VMEM((8, _CH), jnp.float32),
        ],
        compiler_params=pltpu.CompilerParams(
            dimension_semantics=("arbitrary", "arbitrary")),
    )(gath, gath, gath, dkf, dff, W1aT, W1bT, b1r, g1r, be1r, W2T, b2r,
      g2r, be2r)

    return out.reshape(_B, _N, _CH)


# speedup vs baseline: 34.0948x; 1.0251x over previous
import functools

import jax
import jax.numpy as jnp
from jax import lax
from jax.experimental import pallas as pl
from jax.experimental.pallas import tpu as pltpu
from jax.experimental.pallas import tpu_sc as plsc

_B, _N, _M = 4, 8192, 2048
_C11, _C12 = 32, 64
_CH = 64
_BN1 = 1024
_BN2 = 2048
_NPTS = float(_B * _N)
_EPS_BN = 1e-5
_EPS_D = 1e-8

_NC, _NS = 2, 16
_NW = _NC * _NS
_NROWS = 3 * _B * _N
_RPW = _NROWS // _NW
_GCH = 128
_NGCH = _RPW // _GCH


def _knn_kernel(xm2_ref, x2_ref, src_xyzT_ref, s2_ref, dk_ref, gidx_ref):
    d = jnp.dot(xm2_ref[0], src_xyzT_ref[0],
                preferred_element_type=jnp.float32)
    d = d + x2_ref[0]
    d = d + s2_ref[0]

    iota_i = jax.lax.broadcasted_iota(jnp.int32, d.shape, 1)
    key = jax.lax.bitcast_convert_type(iota_i | jnp.int32(0x3F800000),
                                       jnp.float32)
    vals, idxs = [], []
    dd = d
    for k in range(3):
        mval = jnp.min(dd, axis=1, keepdims=True)
        mkey = jnp.min(jnp.where(dd == mval, key, jnp.float32(2.0)),
                       axis=1, keepdims=True)
        vals.append(mval)
        idxs.append(jax.lax.bitcast_convert_type(mkey, jnp.int32)
                    & jnp.int32(0x007FFFFF))
        if k < 2:
            dd = jnp.where(key == mkey, jnp.float32(jnp.inf), dd)
    dk_ref[...] = jnp.concatenate(vals, axis=1)[None]
    base = pl.program_id(0) * _M
    gidx_ref[...] = (jnp.concatenate(idxs, axis=1) + base)[None]


def _sc_gather_body(table_hbm, idx_hbm, out_hbm, idx_v, rows0, gsem0):
    wid = lax.axis_index("s") * _NC + lax.axis_index("c")
    base = wid * _NGCH

    pltpu.sync_copy(idx_hbm.at[pl.ds(base, _NGCH)], idx_v)

    for c in range(_NGCH):
        pltpu.async_copy(table_hbm.at[idx_v.at[c]], rows0, gsem0).wait()
        pltpu.sync_copy(rows0, out_hbm.at[pl.ds((base + c) * _GCH, _GCH)])


def _mlp_fused_kernel(g0_ref, g1_ref, g2_ref, dk_ref, dst_feat_ref,
                      W1aT_ref, W1bT_ref, b1_ref, g1p_ref, be1p_ref,
                      W2T_ref, b2_ref, g2p_ref, be2p_ref,
                      out_ref, y1s, y2s, st1, st2):
    p = pl.program_id(0)
    i = pl.program_id(1)
    rows = pl.ds(i * _BN2, _BN2)

    @pl.when(p == 0)
    def _phase0():
        r = 1.0 / (dk_ref[...] + _EPS_D)
        r0, r1, r2 = r[:, 0:1], r[:, 1:2], r[:, 2:3]
        inv = 1.0 / ((r0 + r1) + r2)
        interp = ((r0 * inv) * g0_ref[...] + (r1 * inv) * g1_ref[...]
                  + (r2 * inv) * g2_ref[...])
        y1 = (jnp.dot(dst_feat_ref[...], W1aT_ref[...],
                      preferred_element_type=jnp.float32)
              + jnp.dot(interp, W1bT_ref[...],
                        preferred_element_type=jnp.float32)
              + b1_ref[...])
        y1s[rows, :] = y1

        @pl.when(i == 0)
        def _init():
            st1[...] = jnp.zeros_like(st1)

        su = jnp.sum(y1, axis=0, keepdims=True)
        sq = jnp.sum(y1 * y1, axis=0, keepdims=True)
        st1[...] += jnp.concatenate(
            [su, sq, jnp.zeros((6, _CH), jnp.float32)], axis=0)

    @pl.when(p == 1)
    def _phase1():
        st = st1[...]
        mean = st[0:1, :] / _NPTS
        var = st[1:2, :] / _NPTS - mean * mean
        y = y1s[rows, :]
        h = jnp.maximum(
            g1p_ref[...] * (y - mean) * jax.lax.rsqrt(var + _EPS_BN)
            + be1p_ref[...], 0.0)
        y2 = jnp.dot(h, W2T_ref[...],
                     preferred_element_type=jnp.float32) + b2_ref[...]
        y2s[rows, :] = y2

        @pl.when(i == 0)
        def _init():
            st2[...] = jnp.zeros_like(st2)

        su = jnp.sum(y2, axis=0, keepdims=True)
        sq = jnp.sum(y2 * y2, axis=0, keepdims=True)
        st2[...] += jnp.concatenate(
            [su, sq, jnp.zeros((6, _CH), jnp.float32)], axis=0)

    @pl.when(p == 2)
    def _phase2():
        st = st2[...]
        mean = st[0:1, :] / _NPTS
        var = st[1:2, :] / _NPTS - mean * mean
        y = y2s[rows, :]
        out_ref[...] = jnp.maximum(
            g2p_ref[...] * (y - mean) * jax.lax.rsqrt(var + _EPS_BN)
            + be2p_ref[...], 0.0)


def kernel(dst_xyz, dst_feat, src_xyz, src_feat, W1, b1, g1, be1, W2, b2, g2, be2):
    src_xyzT = jnp.transpose(src_xyz, (0, 2, 1))
    xm2 = -2.0 * dst_xyz
    x2 = jnp.sum(dst_xyz ** 2, axis=-1, keepdims=True)
    s2 = jnp.sum(src_xyz ** 2, axis=-1)[:, None, :]
    W1aT = jnp.transpose(W1[:, :_C11])
    W1bT = jnp.transpose(W1[:, _C11:])
    W2T = jnp.transpose(W2)
    b1r = b1.reshape(1, _CH)
    b2r = b2.reshape(1, _CH)
    g1r = g1.reshape(1, _CH)
    be1r = be1.reshape(1, _CH)
    g2r = g2.reshape(1, _CH)
    be2r = be2.reshape(1, _CH)

    nb1 = _N // _BN1
    dk, gidx = pl.pallas_call(
        _knn_kernel,
        grid=(_B, nb1),
        in_specs=[
            pl.BlockSpec((1, _BN1, 3), lambda b, i: (b, i, 0)),
            pl.BlockSpec((1, _BN1, 1), lambda b, i: (b, i, 0)),
            pl.BlockSpec((1, 3, _M), lambda b, i: (b, 0, 0)),
            pl.BlockSpec((1, 1, _M), lambda b, i: (b, 0, 0)),
        ],
        out_specs=[
            pl.BlockSpec((1, _BN1, 3), lambda b, i: (b, i, 0)),
            pl.BlockSpec((1, _BN1, 3), lambda b, i: (b, i, 0)),
        ],
        out_shape=[
            jax.ShapeDtypeStruct((_B, _N, 3), jnp.float32),
            jax.ShapeDtypeStruct((_B, _N, 3), jnp.int32),
        ],
        compiler_params=pltpu.CompilerParams(
            dimension_semantics=("arbitrary", "arbitrary")),
    )(xm2, x2, src_xyzT, s2)

    gidx_km = jnp.transpose(gidx, (2, 0, 1)).reshape(_NW * _NGCH, _GCH)
    table = src_feat.reshape(_B * _M, _C12)

    mesh = plsc.VectorSubcoreMesh(core_axis_name="c", subcore_axis_name="s")
    gath = pl.kernel(
        _sc_gather_body,
        out_type=jax.ShapeDtypeStruct((_NROWS, _C12), jnp.float32),
        mesh=mesh,
        scratch_types=[
            pltpu.VMEM((_NGCH, _GCH), jnp.int32),
            pltpu.VMEM((_GCH, _C12), jnp.float32),
            pltpu.SemaphoreType.DMA,
        ],
        compiler_params=pltpu.CompilerParams(use_tc_tiling_on_sc=False),
    )(table, gidx_km)

    dkf = dk.reshape(_B * _N, 3)
    dff = dst_feat.reshape(_B * _N, _C11)
    nb2 = (_B * _N) // _BN2

    def _ph0_map(p, i, off=0):
        return (jnp.where(p == 0, i + off, 0), 0)

    out = pl.pallas_call(
        _mlp_fused_kernel,
        grid=(3, nb2),
        in_specs=[
            pl.BlockSpec((_BN2, _C12), functools.partial(_ph0_map, off=0)),
            pl.BlockSpec((_BN2, _C12), functools.partial(_ph0_map, off=nb2)),
            pl.BlockSpec((_BN2, _C12),
                         functools.partial(_ph0_map, off=2 * nb2)),
            pl.BlockSpec((_BN2, 3), _ph0_map),
            pl.BlockSpec((_BN2, _C11), _ph0_map),
            pl.BlockSpec((_C11, _CH), lambda p, i: (0, 0)),
            pl.BlockSpec((_C12, _CH), lambda p, i: (0, 0)),
            pl.BlockSpec((1, _CH), lambda p, i: (0, 0)),
            pl.BlockSpec((1, _CH), lambda p, i: (0, 0)),
            pl.BlockSpec((1, _CH), lambda p, i: (0, 0)),
            pl.BlockSpec((_CH, _CH), lambda p, i: (0, 0)),
            pl.BlockSpec((1, _CH), lambda p, i: (0, 0)),
            pl.BlockSpec((1, _CH), lambda p, i: (0, 0)),
            pl.BlockSpec((1, _CH), lambda p, i: (0, 0)),
        ],
        out_specs=pl.BlockSpec((_BN2, _CH),
                               lambda p, i: (jnp.where(p == 2, i, 0), 0)),
        out_shape=jax.ShapeDtypeStruct((_B * _N, _CH), jnp.float32),
        scratch_shapes=[
            pltpu.VMEM((_B * _N, _CH), jnp.float32),
            pltpu.VMEM((_B * _N, _CH), jnp.float32),
            pltpu.VMEM((8, _CH), jnp.float32),
            pltpu.VMEM((8, _CH), jnp.float32),
        ],
        compiler_params=pltpu.CompilerParams(
            dimension_semantics=("arbitrary", "arbitrary")),
    )(gath, gath, gath, dkf, dff, W1aT, W1bT, b1r, g1r, be1r, W2T, b2r,
      g2r, be2r)

    return out.reshape(_B, _N, _CH)


# speedup vs baseline: 34.6181x; 1.0153x over previous
import functools

import jax
import jax.numpy as jnp
from jax import lax
from jax.experimental import pallas as pl
from jax.experimental.pallas import tpu as pltpu
from jax.experimental.pallas import tpu_sc as plsc

_B, _N, _M = 4, 8192, 2048
_C11, _C12 = 32, 64
_CH = 64
_BN1 = 1024
_BN2 = 2048
_NPTS = float(_B * _N)
_EPS_BN = 1e-5
_EPS_D = 1e-8

_NC, _NS = 2, 16
_NW = _NC * _NS
_NROWS = 3 * _B * _N
_RPW = _NROWS // _NW
_GCH = 128
_NGCH = _RPW // _GCH


def _knn_kernel(xm2_ref, x2_ref, src_xyzT_ref, s2_ref, dk_ref, gidx_ref):
    d = jnp.dot(xm2_ref[0], src_xyzT_ref[0],
                preferred_element_type=jnp.float32)
    d = d + x2_ref[0]
    d = d + s2_ref[0]

    iota_i = jax.lax.broadcasted_iota(jnp.int32, d.shape, 1)
    key = jax.lax.bitcast_convert_type(iota_i | jnp.int32(0x3F800000),
                                       jnp.float32)
    vals, idxs = [], []
    dd = d
    for k in range(3):
        mval = jnp.min(dd, axis=1, keepdims=True)
        mkey = jnp.min(jnp.where(dd == mval, key, jnp.float32(2.0)),
                       axis=1, keepdims=True)
        vals.append(mval)
        idxs.append(jax.lax.bitcast_convert_type(mkey, jnp.int32)
                    & jnp.int32(0x007FFFFF))
        if k < 2:
            dd = jnp.where(key == mkey, jnp.float32(jnp.inf), dd)
    dk_ref[...] = jnp.concatenate(vals, axis=1)[None]
    base = pl.program_id(0) * _M
    gidx_ref[...] = (jnp.concatenate(idxs, axis=1) + base)[None]


def _sc_gather_body(table_hbm, idx_hbm, out_hbm, idx_v, rows0, rows1,
                    gsem0, gsem1):
    wid = lax.axis_index("s") * _NC + lax.axis_index("c")
    base = wid * _NGCH
    rows = (rows0, rows1)
    gsems = (gsem0, gsem1)

    pltpu.sync_copy(idx_hbm.at[pl.ds(base, _NGCH)], idx_v)

    handles = [None, None]
    handles[0] = pltpu.async_copy(table_hbm.at[idx_v.at[0]], rows[0], gsems[0])
    for c in range(_NGCH):
        cur = c & 1
        nxt = cur ^ 1
        handles[cur].wait()
        if c + 1 < _NGCH:
            handles[nxt] = pltpu.async_copy(table_hbm.at[idx_v.at[c + 1]],
                                            rows[nxt], gsems[nxt])
        pltpu.sync_copy(rows[cur], out_hbm.at[pl.ds((base + c) * _GCH, _GCH)])


def _mlp_fused_kernel(g0_ref, g1_ref, g2_ref, dk_ref, dst_feat_ref,
                      W1aT_ref, W1bT_ref, b1_ref, g1p_ref, be1p_ref,
                      W2T_ref, b2_ref, g2p_ref, be2p_ref,
                      out_ref, y1s, y2s, st1, st2):
    p = pl.program_id(0)
    i = pl.program_id(1)
    rows = pl.ds(i * _BN2, _BN2)

    @pl.when(p == 0)
    def _phase0():
        r = 1.0 / (dk_ref[...] + _EPS_D)
        r0, r1, r2 = r[:, 0:1], r[:, 1:2], r[:, 2:3]
        inv = 1.0 / ((r0 + r1) + r2)
        interp = ((r0 * inv) * g0_ref[...] + (r1 * inv) * g1_ref[...]
                  + (r2 * inv) * g2_ref[...])
        y1 = (jnp.dot(dst_feat_ref[...], W1aT_ref[...],
                      preferred_element_type=jnp.float32)
              + jnp.dot(interp, W1bT_ref[...],
                        preferred_element_type=jnp.float32)
              + b1_ref[...])
        y1s[rows, :] = y1

        @pl.when(i == 0)
        def _init():
            st1[...] = jnp.zeros_like(st1)

        su = jnp.sum(y1, axis=0, keepdims=True)
        sq = jnp.sum(y1 * y1, axis=0, keepdims=True)
        st1[...] += jnp.concatenate(
            [su, sq, jnp.zeros((6, _CH), jnp.float32)], axis=0)

    @pl.when(p == 1)
    def _phase1():
        st = st1[...]
        mean = st[0:1, :] / _NPTS
        var = st[1:2, :] / _NPTS - mean * mean
        y = y1s[rows, :]
        h = jnp.maximum(
            g1p_ref[...] * (y - mean) * jax.lax.rsqrt(var + _EPS_BN)
            + be1p_ref[...], 0.0)
        y2 = jnp.dot(h, W2T_ref[...],
                     preferred_element_type=jnp.float32) + b2_ref[...]
        y2s[rows, :] = y2

        @pl.when(i == 0)
        def _init():
            st2[...] = jnp.zeros_like(st2)

        su = jnp.sum(y2, axis=0, keepdims=True)
        sq = jnp.sum(y2 * y2, axis=0, keepdims=True)
        st2[...] += jnp.concatenate(
            [su, sq, jnp.zeros((6, _CH), jnp.float32)], axis=0)

    @pl.when(p == 2)
    def _phase2():
        st = st2[...]
        mean = st[0:1, :] / _NPTS
        var = st[1:2, :] / _NPTS - mean * mean
        y = y2s[rows, :]
        out_ref[...] = jnp.maximum(
            g2p_ref[...] * (y - mean) * jax.lax.rsqrt(var + _EPS_BN)
            + be2p_ref[...], 0.0)


def kernel(dst_xyz, dst_feat, src_xyz, src_feat, W1, b1, g1, be1, W2, b2, g2, be2):
    src_xyzT = jnp.transpose(src_xyz, (0, 2, 1))
    xm2 = -2.0 * dst_xyz
    x2 = jnp.sum(dst_xyz ** 2, axis=-1, keepdims=True)
    s2 = jnp.sum(src_xyz ** 2, axis=-1)[:, None, :]
    W1aT = jnp.transpose(W1[:, :_C11])
    W1bT = jnp.transpose(W1[:, _C11:])
    W2T = jnp.transpose(W2)
    b1r = b1.reshape(1, _CH)
    b2r = b2.reshape(1, _CH)
    g1r = g1.reshape(1, _CH)
    be1r = be1.reshape(1, _CH)
    g2r = g2.reshape(1, _CH)
    be2r = be2.reshape(1, _CH)

    nb1 = _N // _BN1
    dk, gidx = pl.pallas_call(
        _knn_kernel,
        grid=(_B, nb1),
        in_specs=[
            pl.BlockSpec((1, _BN1, 3), lambda b, i: (b, i, 0)),
            pl.BlockSpec((1, _BN1, 1), lambda b, i: (b, i, 0)),
            pl.BlockSpec((1, 3, _M), lambda b, i: (b, 0, 0)),
            pl.BlockSpec((1, 1, _M), lambda b, i: (b, 0, 0)),
        ],
        out_specs=[
            pl.BlockSpec((1, _BN1, 3), lambda b, i: (b, i, 0)),
            pl.BlockSpec((1, _BN1, 3), lambda b, i: (b, i, 0)),
        ],
        out_shape=[
            jax.ShapeDtypeStruct((_B, _N, 3), jnp.float32),
            jax.ShapeDtypeStruct((_B, _N, 3), jnp.int32),
        ],
        compiler_params=pltpu.CompilerParams(
            dimension_semantics=("arbitrary", "arbitrary")),
    )(xm2, x2, src_xyzT, s2)

    gidx_km = jnp.transpose(gidx, (2, 0, 1)).reshape(_NW * _NGCH, _GCH)
    table = src_feat.reshape(_B * _M, _C12)

    mesh = plsc.VectorSubcoreMesh(core_axis_name="c", subcore_axis_name="s")
    gath = pl.kernel(
        _sc_gather_body,
        out_type=jax.ShapeDtypeStruct((_NROWS, _C12), jnp.float32),
        mesh=mesh,
        scratch_types=[
            pltpu.VMEM((_NGCH, _GCH), jnp.int32),
            pltpu.VMEM((_GCH, _C12), jnp.float32),
            pltpu.VMEM((_GCH, _C12), jnp.float32),
            pltpu.SemaphoreType.DMA,
            pltpu.SemaphoreType.DMA,
        ],
        compiler_params=pltpu.CompilerParams(use_tc_tiling_on_sc=False),
    )(table, gidx_km)

    dkf = dk.reshape(_B * _N, 3)
    dff = dst_feat.reshape(_B * _N, _C11)
    nb2 = (_B * _N) // _BN2

    def _ph0_map(p, i, off=0):
        return (jnp.where(p == 0, i + off, 0), 0)

    out = pl.pallas_call(
        _mlp_fused_kernel,
        grid=(3, nb2),
        in_specs=[
            pl.BlockSpec((_BN2, _C12), functools.partial(_ph0_map, off=0)),
            pl.BlockSpec((_BN2, _C12), functools.partial(_ph0_map, off=nb2)),
            pl.BlockSpec((_BN2, _C12),
                         functools.partial(_ph0_map, off=2 * nb2)),
            pl.BlockSpec((_BN2, 3), _ph0_map),
            pl.BlockSpec((_BN2, _C11), _ph0_map),
            pl.BlockSpec((_C11, _CH), lambda p, i: (0, 0)),
            pl.BlockSpec((_C12, _CH), lambda p, i: (0, 0)),
            pl.BlockSpec((1, _CH), lambda p, i: (0, 0)),
            pl.BlockSpec((1, _CH), lambda p, i: (0, 0)),
            pl.BlockSpec((1, _CH), lambda p, i: (0, 0)),
            pl.BlockSpec((_CH, _CH), lambda p, i: (0, 0)),
            pl.BlockSpec((1, _CH), lambda p, i: (0, 0)),
            pl.BlockSpec((1, _CH), lambda p, i: (0, 0)),
            pl.BlockSpec((1, _CH), lambda p, i: (0, 0)),
        ],
        out_specs=pl.BlockSpec((_BN2, _CH),
                               lambda p, i: (jnp.where(p == 2, i, 0), 0)),
        out_shape=jax.ShapeDtypeStruct((_B * _N, _CH), jnp.float32),
        scratch_shapes=[
            pltpu.VMEM((_B * _N, _CH), jnp.float32),
            pltpu.VMEM((_B * _N, _CH), jnp.float32),
            pltpu.VMEM((8, _CH), jnp.float32),
            pltpu.VMEM((8, _CH), jnp.float32),
        ],
        compiler_params=pltpu.CompilerParams(
            dimension_semantics=("arbitrary", "arbitrary")),
    )(gath, gath, gath, dkf, dff, W1aT, W1bT, b1r, g1r, be1r, W2T, b2r,
      g2r, be2r)

    return out.reshape(_B, _N, _CH)
